# baseline (speedup 1.0000x reference)
import functools

import jax
import jax.numpy as jnp
from jax import lax
from jax.experimental import pallas as pl
from jax.experimental.pallas import tpu as pltpu
from jax.experimental.pallas import tpu_sc as plsc

_BETA = 0.25
_ROW_TILE = 512
_K_TILE = 2048
_LOSS_TILE = 1024


def _scan_kernel(x_ref, we_ref, v_ref, z_ref, idx_ref, va_ref):
    K, C = v_ref.shape
    n_kt = K // _K_TILE
    rows = x_ref.shape[0]

    @pl.when(pl.program_id(0) == 0)
    def _prep():
        v = v_ref[...]
        va_ref[:, :C] = v + v
        va_ref[:, C:] = -jnp.sum(v * v, axis=1, keepdims=True)

    x = x_ref[...]
    z = jnp.dot(x, we_ref[...], preferred_element_type=jnp.float32)
    z_ref[...] = z
    z1 = jnp.concatenate([z, jnp.ones((rows, 1), jnp.float32)], axis=1)

    def scan_body(j, run):
        vat = va_ref[pl.ds(j * _K_TILE, _K_TILE), :]
        score = lax.dot_general(
            z1, vat, (((1,), (1,)), ((), ())),
            preferred_element_type=jnp.float32)
        enc = (K - 1 - j * _K_TILE) - lax.broadcasted_iota(
            jnp.int32, (1, _K_TILE), 1)
        si = lax.bitcast_convert_type(score, jnp.int32)
        packed = jnp.bitwise_or(jnp.bitwise_and(si, -8192), enc)
        pf = lax.bitcast_convert_type(packed, jnp.float32)
        return jnp.maximum(run, jnp.max(pf, axis=1, keepdims=True))

    run = lax.fori_loop(0, n_kt, scan_body,
                        jnp.full((rows, 1), -jnp.inf, jnp.float32))
    pi = lax.bitcast_convert_type(run, jnp.int32)
    best_idx = (K - 1) - jnp.bitwise_and(pi, 8191)
    idx_ref[...] = best_idx.reshape(1, 1, rows)


def _sc_gather(vectors_padded, idx):
    K, C = vectors_padded.shape
    B = idx.shape[0]
    info = plsc.get_sparse_core_info()
    nw = info.num_cores * info.num_subcores
    b_per_w = B // nw
    mesh = plsc.VectorSubcoreMesh(core_axis_name="c", subcore_axis_name="s")

    @functools.partial(
        pl.kernel, mesh=mesh,
        out_type=jax.ShapeDtypeStruct((B, C), jnp.float32),
        scratch_types=[
            pltpu.VMEM((b_per_w,), jnp.int32),
            pltpu.VMEM((b_per_w, C), jnp.float32),
            pltpu.SemaphoreType.DMA,
        ],
    )
    def k(table_hbm, idx_hbm, out_hbm, idx_v, rows_v, sem):
        wid = lax.axis_index("s") * info.num_cores + lax.axis_index("c")
        base = wid * b_per_w
        pltpu.sync_copy(idx_hbm.at[pl.ds(base, b_per_w)], idx_v)
        pltpu.async_copy(table_hbm.at[idx_v], rows_v, sem).wait()
        pltpu.sync_copy(rows_v, out_hbm.at[pl.ds(base, b_per_w)])

    return k(vectors_padded, idx)


def _loss_kernel(x_ref, z_ref, q_ref, wd_ref, b_ref, recon_ref, commit_ref):
    x = x_ref[...]
    z = z_ref[...]
    q = q_ref[:, :z_ref.shape[1]]
    z_q = z + (q - z)
    mean = jnp.dot(z_q, wd_ref[...], preferred_element_type=jnp.float32)
    mean = mean + b_ref[...]
    r = x - mean
    dq = q - z
    recon_part = jnp.sum(r * r)
    commit_part = jnp.sum(dq * dq)

    @pl.when(pl.program_id(0) == 0)
    def _init():
        recon_ref[0, 0] = recon_part
        commit_ref[0, 0] = commit_part

    @pl.when(pl.program_id(0) != 0)
    def _acc():
        recon_ref[0, 0] += recon_part
        commit_ref[0, 0] += commit_part


@jax.jit
def kernel(x, W_enc, vectors, W_dec, b_dec):
    B, T, D_IN = x.shape
    K, D_CODE = vectors.shape
    rows = B * T
    x2 = x.reshape(rows, D_IN)
    b2 = b_dec.reshape(1, D_IN)
    n_scan_tiles = rows // _ROW_TILE

    z_flat, idx3 = pl.pallas_call(
        _scan_kernel,
        grid=(n_scan_tiles,),
        in_specs=[
            pl.BlockSpec((_ROW_TILE, D_IN), lambda i: (i, 0)),
            pl.BlockSpec((D_IN, D_CODE), lambda i: (0, 0)),
            pl.BlockSpec((K, D_CODE), lambda i: (0, 0)),
        ],
        out_specs=[
            pl.BlockSpec((_ROW_TILE, D_CODE), lambda i: (i, 0)),
            pl.BlockSpec((1, 1, _ROW_TILE), lambda i: (i, 0, 0)),
        ],
        out_shape=[
            jax.ShapeDtypeStruct((rows, D_CODE), jnp.float32),
            jax.ShapeDtypeStruct((n_scan_tiles, 1, _ROW_TILE), jnp.int32),
        ],
        scratch_shapes=[
            pltpu.VMEM((K, D_CODE + 1), jnp.float32),
        ],
    )(x2, W_enc, vectors)

    v_padded = jnp.pad(vectors, ((0, 0), (0, 128 - D_CODE)))
    q_padded = _sc_gather(v_padded, idx3.reshape(rows))

    recon_sum, commit_sum = pl.pallas_call(
        _loss_kernel,
        grid=(rows // _LOSS_TILE,),
        in_specs=[
            pl.BlockSpec((_LOSS_TILE, D_IN), lambda i: (i, 0)),
            pl.BlockSpec((_LOSS_TILE, D_CODE), lambda i: (i, 0)),
            pl.BlockSpec((_LOSS_TILE, 128), lambda i: (i, 0)),
            pl.BlockSpec((D_CODE, D_IN), lambda i: (0, 0)),
            pl.BlockSpec((1, D_IN), lambda i: (0, 0)),
        ],
        out_specs=[
            pl.BlockSpec(memory_space=pltpu.SMEM),
            pl.BlockSpec(memory_space=pltpu.SMEM),
        ],
        out_shape=[
            jax.ShapeDtypeStruct((1, 1), jnp.float32),
            jax.ShapeDtypeStruct((1, 1), jnp.float32),
        ],
    )(x2, z_flat, q_padded, W_dec, b2)

    recon = 0.5 * recon_sum[0, 0] / rows + 0.5 * D_IN * jnp.log(2.0 * jnp.pi)
    commit = commit_sum[0, 0] / (rows * D_CODE)
    return recon + _BETA * commit


# speedup vs baseline: 1.7514x; 1.7514x over previous
import jax
import jax.numpy as jnp
from jax import lax
from jax.experimental import pallas as pl
from jax.experimental.pallas import tpu as pltpu

_BETA = 0.25
_ROW_TILE = 512
_K_TILE = 2048
_HI = 128


def _vq_kernel(x_ref, we_ref, v_ref, v128_ref, wd_ref, b_ref,
               recon_ref, commit_ref, va_ref):
    K, C = v_ref.shape
    n_kt = K // _K_TILE
    rows = x_ref.shape[0]
    lo_width = K // _HI

    @pl.when(pl.program_id(0) == 0)
    def _prep():
        v = v_ref[...]
        va_ref[:, :C] = v + v
        va_ref[:, C:] = -jnp.sum(v * v, axis=1, keepdims=True)

    x = x_ref[...]
    z = jnp.dot(x, we_ref[...], preferred_element_type=jnp.float32)
    z1 = jnp.concatenate([z, jnp.ones((rows, 1), jnp.float32)], axis=1)

    def scan_body(j, run):
        vat = va_ref[pl.ds(j * _K_TILE, _K_TILE), :]
        score = lax.dot_general(
            z1, vat, (((1,), (1,)), ((), ())),
            preferred_element_type=jnp.float32)
        enc = (K - 1 - j * _K_TILE) - lax.broadcasted_iota(
            jnp.int32, (1, _K_TILE), 1)
        si = lax.bitcast_convert_type(score, jnp.int32)
        packed = jnp.bitwise_or(jnp.bitwise_and(si, -8192), enc)
        pf = lax.bitcast_convert_type(packed, jnp.float32)
        return jnp.maximum(run, jnp.max(pf, axis=1, keepdims=True))

    run = lax.fori_loop(0, n_kt, scan_body,
                        jnp.full((rows, 1), -jnp.inf, jnp.float32))
    pi = lax.bitcast_convert_type(run, jnp.int32)
    best_idx = (K - 1) - jnp.bitwise_and(pi, 8191)

    hi = best_idx >> 6
    lo = best_idx & (lo_width - 1)
    onehot_hi = (hi == lax.broadcasted_iota(jnp.int32, (1, _HI), 1)
                 ).astype(jnp.float32)
    block = jnp.dot(onehot_hi, v128_ref[...],
                    preferred_element_type=jnp.float32)
    grp = lax.broadcasted_iota(jnp.int32, (1, lo_width * C), 1) // C
    masked = jnp.where(grp == lo, block, 0.0)
    w = lo_width * C
    while w > C:
        w //= 2
        masked = masked[:, :w] + masked[:, w:2 * w]
    q = masked

    z_q = z + (q - z)
    mean = jnp.dot(z_q, wd_ref[...], preferred_element_type=jnp.float32)
    mean = mean + b_ref[...]
    r = x - mean
    dq = q - z
    recon_part = jnp.sum(r * r)
    commit_part = jnp.sum(dq * dq)

    @pl.when(pl.program_id(0) == 0)
    def _init():
        recon_ref[0, 0] = recon_part
        commit_ref[0, 0] = commit_part

    @pl.when(pl.program_id(0) != 0)
    def _acc():
        recon_ref[0, 0] += recon_part
        commit_ref[0, 0] += commit_part


@jax.jit
def kernel(x, W_enc, vectors, W_dec, b_dec):
    B, T, D_IN = x.shape
    K, D_CODE = vectors.shape
    rows = B * T
    x2 = x.reshape(rows, D_IN)
    b2 = b_dec.reshape(1, D_IN)
    v128 = vectors.reshape(_HI, (K // _HI) * D_CODE)
    n_row_tiles = rows // _ROW_TILE

    recon_sum, commit_sum = pl.pallas_call(
        _vq_kernel,
        grid=(n_row_tiles,),
        in_specs=[
            pl.BlockSpec((_ROW_TILE, D_IN), lambda i: (i, 0)),
            pl.BlockSpec((D_IN, D_CODE), lambda i: (0, 0)),
            pl.BlockSpec((K, D_CODE), lambda i: (0, 0)),
            pl.BlockSpec(v128.shape, lambda i: (0, 0)),
            pl.BlockSpec((D_CODE, D_IN), lambda i: (0, 0)),
            pl.BlockSpec((1, D_IN), lambda i: (0, 0)),
        ],
        out_specs=[
            pl.BlockSpec(memory_space=pltpu.SMEM),
            pl.BlockSpec(memory_space=pltpu.SMEM),
        ],
        out_shape=[
            jax.ShapeDtypeStruct((1, 1), jnp.float32),
            jax.ShapeDtypeStruct((1, 1), jnp.float32),
        ],
        scratch_shapes=[
            pltpu.VMEM((K, D_CODE + 1), jnp.float32),
        ],
    )(x2, W_enc, vectors, v128, W_dec, b2)

    recon = 0.5 * recon_sum[0, 0] / rows + 0.5 * D_IN * jnp.log(2.0 * jnp.pi)
    commit = commit_sum[0, 0] / (rows * D_CODE)
    return recon + _BETA * commit


# speedup vs baseline: 1.8550x; 1.0591x over previous
import jax
import jax.numpy as jnp
from jax import lax
from jax.experimental import pallas as pl
from jax.experimental.pallas import tpu as pltpu

_BETA = 0.25
_ROW_TILE = 512
_K_TILE = 2048
_HI = 512


def _vq_kernel(x_ref, we_ref, v_ref, v128_ref, wd_ref, b_ref,
               recon_ref, commit_ref, va_ref):
    K, C = v_ref.shape
    n_kt = K // _K_TILE
    rows = x_ref.shape[0]
    lo_width = K // _HI

    @pl.when(pl.program_id(0) == 0)
    def _prep():
        v = v_ref[...]
        va_ref[:, :C] = v + v
        va_ref[:, C:] = -jnp.sum(v * v, axis=1, keepdims=True)

    x = x_ref[...]
    z = jnp.dot(x, we_ref[...], preferred_element_type=jnp.float32)
    z1 = jnp.concatenate([z, jnp.ones((rows, 1), jnp.float32)], axis=1)

    iota_kt = lax.broadcasted_iota(jnp.int32, (1, _K_TILE), 1)

    def scan_body(j, run):
        vat = va_ref[pl.ds(j * _K_TILE, _K_TILE), :]
        score = lax.dot_general(
            z1, vat, (((1,), (1,)), ((), ())),
            preferred_element_type=jnp.float32)
        enc = (K - 1 - j * _K_TILE) - iota_kt
        si = lax.bitcast_convert_type(score, jnp.int32)
        packed = jnp.bitwise_or(jnp.bitwise_and(si, -8192), enc)
        pf = lax.bitcast_convert_type(packed, jnp.float32)
        return jnp.maximum(run, jnp.max(pf, axis=1, keepdims=True))

    run = lax.fori_loop(0, n_kt, scan_body,
                        jnp.full((rows, 1), -jnp.inf, jnp.float32))
    pi = lax.bitcast_convert_type(run, jnp.int32)
    best_idx = (K - 1) - jnp.bitwise_and(pi, 8191)

    lo_shift = lo_width.bit_length() - 1
    hi = best_idx >> lo_shift
    lo = best_idx & (lo_width - 1)
    onehot_hi = (hi == lax.broadcasted_iota(jnp.int32, (1, _HI), 1)
                 ).astype(jnp.float32)
    block = jnp.dot(onehot_hi, v128_ref[...],
                    preferred_element_type=jnp.float32)
    grp = lax.broadcasted_iota(jnp.int32, (1, lo_width * C), 1) // C
    masked = jnp.where(grp == lo, block, 0.0)
    w = lo_width * C
    while w > C:
        w //= 2
        masked = masked[:, :w] + masked[:, w:2 * w]
    q = masked

    z_q = z + (q - z)
    mean = jnp.dot(z_q, wd_ref[...], preferred_element_type=jnp.float32)
    mean = mean + b_ref[...]
    r = x - mean
    dq = q - z
    recon_part = jnp.sum(r * r)
    commit_part = jnp.sum(dq * dq)

    @pl.when(pl.program_id(0) == 0)
    def _init():
        recon_ref[0, 0] = recon_part
        commit_ref[0, 0] = commit_part

    @pl.when(pl.program_id(0) != 0)
    def _acc():
        recon_ref[0, 0] += recon_part
        commit_ref[0, 0] += commit_part


@jax.jit
def kernel(x, W_enc, vectors, W_dec, b_dec):
    B, T, D_IN = x.shape
    K, D_CODE = vectors.shape
    rows = B * T
    x2 = x.reshape(rows, D_IN)
    b2 = b_dec.reshape(1, D_IN)
    v128 = vectors.reshape(_HI, (K // _HI) * D_CODE)
    n_row_tiles = rows // _ROW_TILE

    recon_sum, commit_sum = pl.pallas_call(
        _vq_kernel,
        grid=(n_row_tiles,),
        in_specs=[
            pl.BlockSpec((_ROW_TILE, D_IN), lambda i: (i, 0)),
            pl.BlockSpec((D_IN, D_CODE), lambda i: (0, 0)),
            pl.BlockSpec((K, D_CODE), lambda i: (0, 0)),
            pl.BlockSpec(v128.shape, lambda i: (0, 0)),
            pl.BlockSpec((D_CODE, D_IN), lambda i: (0, 0)),
            pl.BlockSpec((1, D_IN), lambda i: (0, 0)),
        ],
        out_specs=[
            pl.BlockSpec(memory_space=pltpu.SMEM),
            pl.BlockSpec(memory_space=pltpu.SMEM),
        ],
        out_shape=[
            jax.ShapeDtypeStruct((1, 1), jnp.float32),
            jax.ShapeDtypeStruct((1, 1), jnp.float32),
        ],
        scratch_shapes=[
            pltpu.VMEM((K, D_CODE + 1), jnp.float32),
        ],
    )(x2, W_enc, vectors, v128, W_dec, b2)

    recon = 0.5 * recon_sum[0, 0] / rows + 0.5 * D_IN * jnp.log(2.0 * jnp.pi)
    commit = commit_sum[0, 0] / (rows * D_CODE)
    return recon + _BETA * commit


# speedup vs baseline: 2.1473x; 1.1576x over previous
import jax
import jax.numpy as jnp
from jax import lax
from jax.experimental import pallas as pl
from jax.experimental.pallas import tpu as pltpu

_BETA = 0.25
_ROW_TILE = 1024
_K_TILE = 2048
_HI = 512


def _vq_kernel(x_ref, we_ref, v_ref, v128_ref, wd_ref, b_ref,
               recon_ref, commit_ref, va_ref):
    K, C = v_ref.shape
    n_kt = K // _K_TILE
    rows = x_ref.shape[0]
    lo_width = K // _HI

    @pl.when(pl.program_id(0) == 0)
    def _prep():
        v = v_ref[...]
        va_ref[:, :C] = v + v
        va_ref[:, C:] = -jnp.sum(v * v, axis=1, keepdims=True)

    x = x_ref[...]
    z = jnp.dot(x, we_ref[...], preferred_element_type=jnp.float32)
    z1 = jnp.concatenate([z, jnp.ones((rows, 1), jnp.float32)], axis=1)

    iota_kt = lax.broadcasted_iota(jnp.int32, (1, _K_TILE), 1)

    def scan_body(j, run):
        vat = va_ref[pl.ds(j * _K_TILE, _K_TILE), :]
        score = lax.dot_general(
            z1, vat, (((1,), (1,)), ((), ())),
            preferred_element_type=jnp.float32)
        enc = (K - 1 - j * _K_TILE) - iota_kt
        si = lax.bitcast_convert_type(score, jnp.int32)
        packed = jnp.bitwise_or(jnp.bitwise_and(si, -8192), enc)
        pf = lax.bitcast_convert_type(packed, jnp.float32)
        return jnp.maximum(run, jnp.max(pf, axis=1, keepdims=True))

    run = lax.fori_loop(0, n_kt, scan_body,
                        jnp.full((rows, 1), -jnp.inf, jnp.float32))
    pi = lax.bitcast_convert_type(run, jnp.int32)
    best_idx = (K - 1) - jnp.bitwise_and(pi, 8191)

    lo_shift = lo_width.bit_length() - 1
    hi = best_idx >> lo_shift
    lo = best_idx & (lo_width - 1)
    onehot_hi = (hi == lax.broadcasted_iota(jnp.int32, (1, _HI), 1)
                 ).astype(jnp.float32)
    block = jnp.dot(onehot_hi, v128_ref[...],
                    preferred_element_type=jnp.float32)
    grp = lax.broadcasted_iota(jnp.int32, (1, lo_width * C), 1) // C
    masked = jnp.where(grp == lo, block, 0.0)
    w = lo_width * C
    while w > C:
        w //= 2
        masked = masked[:, :w] + masked[:, w:2 * w]
    q = masked

    z_q = z + (q - z)
    mean = jnp.dot(z_q, wd_ref[...], preferred_element_type=jnp.float32)
    mean = mean + b_ref[...]
    r = x - mean
    dq = q - z
    recon_part = jnp.sum(r * r)
    commit_part = jnp.sum(dq * dq)

    @pl.when(pl.program_id(0) == 0)
    def _init():
        recon_ref[0, 0] = recon_part
        commit_ref[0, 0] = commit_part

    @pl.when(pl.program_id(0) != 0)
    def _acc():
        recon_ref[0, 0] += recon_part
        commit_ref[0, 0] += commit_part


@jax.jit
def kernel(x, W_enc, vectors, W_dec, b_dec):
    B, T, D_IN = x.shape
    K, D_CODE = vectors.shape
    rows = B * T
    x2 = x.reshape(rows, D_IN)
    b2 = b_dec.reshape(1, D_IN)
    v128 = vectors.reshape(_HI, (K // _HI) * D_CODE)
    n_row_tiles = rows // _ROW_TILE

    recon_sum, commit_sum = pl.pallas_call(
        _vq_kernel,
        grid=(n_row_tiles,),
        in_specs=[
            pl.BlockSpec((_ROW_TILE, D_IN), lambda i: (i, 0)),
            pl.BlockSpec((D_IN, D_CODE), lambda i: (0, 0)),
            pl.BlockSpec((K, D_CODE), lambda i: (0, 0)),
            pl.BlockSpec(v128.shape, lambda i: (0, 0)),
            pl.BlockSpec((D_CODE, D_IN), lambda i: (0, 0)),
            pl.BlockSpec((1, D_IN), lambda i: (0, 0)),
        ],
        out_specs=[
            pl.BlockSpec(memory_space=pltpu.SMEM),
            pl.BlockSpec(memory_space=pltpu.SMEM),
        ],
        out_shape=[
            jax.ShapeDtypeStruct((1, 1), jnp.float32),
            jax.ShapeDtypeStruct((1, 1), jnp.float32),
        ],
        scratch_shapes=[
            pltpu.VMEM((K, D_CODE + 1), jnp.float32),
        ],
    )(x2, W_enc, vectors, v128, W_dec, b2)

    recon = 0.5 * recon_sum[0, 0] / rows + 0.5 * D_IN * jnp.log(2.0 * jnp.pi)
    commit = commit_sum[0, 0] / (rows * D_CODE)
    return recon + _BETA * commit


# speedup vs baseline: 2.3296x; 1.0849x over previous
import jax
import jax.numpy as jnp
from jax import lax
from jax.experimental import pallas as pl
from jax.experimental.pallas import tpu as pltpu

_BETA = 0.25
_ROW_TILE = 2048
_K_TILE = 2048
_HI = 512


def _vq_kernel(x_ref, we_ref, v_ref, v128_ref, wd_ref, b_ref,
               recon_ref, commit_ref, va_ref):
    K, C = v_ref.shape
    n_kt = K // _K_TILE
    rows = x_ref.shape[0]
    lo_width = K // _HI

    @pl.when(pl.program_id(0) == 0)
    def _prep():
        v = v_ref[...]
        va_ref[:, :C] = v + v
        va_ref[:, C:] = -jnp.sum(v * v, axis=1, keepdims=True)

    x = x_ref[...]
    z = jnp.dot(x, we_ref[...], preferred_element_type=jnp.float32)
    z1 = jnp.concatenate([z, jnp.ones((rows, 1), jnp.float32)], axis=1)

    iota_kt = lax.broadcasted_iota(jnp.int32, (1, _K_TILE), 1)

    def scan_body(j, run):
        vat = va_ref[pl.ds(j * _K_TILE, _K_TILE), :]
        score = lax.dot_general(
            z1, vat, (((1,), (1,)), ((), ())),
            preferred_element_type=jnp.float32)
        enc = (K - 1 - j * _K_TILE) - iota_kt
        si = lax.bitcast_convert_type(score, jnp.int32)
        packed = jnp.bitwise_or(jnp.bitwise_and(si, -8192), enc)
        pf = lax.bitcast_convert_type(packed, jnp.float32)
        return jnp.maximum(run, jnp.max(pf, axis=1, keepdims=True))

    run = lax.fori_loop(0, n_kt, scan_body,
                        jnp.full((rows, 1), -jnp.inf, jnp.float32))
    pi = lax.bitcast_convert_type(run, jnp.int32)
    best_idx = (K - 1) - jnp.bitwise_and(pi, 8191)

    lo_shift = lo_width.bit_length() - 1
    hi = best_idx >> lo_shift
    lo = best_idx & (lo_width - 1)
    onehot_hi = (hi == lax.broadcasted_iota(jnp.int32, (1, _HI), 1)
                 ).astype(jnp.float32)
    block = jnp.dot(onehot_hi, v128_ref[...],
                    preferred_element_type=jnp.float32)
    grp = lax.broadcasted_iota(jnp.int32, (1, lo_width * C), 1) // C
    masked = jnp.where(grp == lo, block, 0.0)
    w = lo_width * C
    while w > C:
        w //= 2
        masked = masked[:, :w] + masked[:, w:2 * w]
    q = masked

    z_q = z + (q - z)
    mean = jnp.dot(z_q, wd_ref[...], preferred_element_type=jnp.float32)
    mean = mean + b_ref[...]
    r = x - mean
    dq = q - z
    recon_part = jnp.sum(r * r)
    commit_part = jnp.sum(dq * dq)

    @pl.when(pl.program_id(0) == 0)
    def _init():
        recon_ref[0, 0] = recon_part
        commit_ref[0, 0] = commit_part

    @pl.when(pl.program_id(0) != 0)
    def _acc():
        recon_ref[0, 0] += recon_part
        commit_ref[0, 0] += commit_part


@jax.jit
def kernel(x, W_enc, vectors, W_dec, b_dec):
    B, T, D_IN = x.shape
    K, D_CODE = vectors.shape
    rows = B * T
    x2 = x.reshape(rows, D_IN)
    b2 = b_dec.reshape(1, D_IN)
    v128 = vectors.reshape(_HI, (K // _HI) * D_CODE)
    n_row_tiles = rows // _ROW_TILE

    recon_sum, commit_sum = pl.pallas_call(
        _vq_kernel,
        grid=(n_row_tiles,),
        in_specs=[
            pl.BlockSpec((_ROW_TILE, D_IN), lambda i: (i, 0)),
            pl.BlockSpec((D_IN, D_CODE), lambda i: (0, 0)),
            pl.BlockSpec((K, D_CODE), lambda i: (0, 0)),
            pl.BlockSpec(v128.shape, lambda i: (0, 0)),
            pl.BlockSpec((D_CODE, D_IN), lambda i: (0, 0)),
            pl.BlockSpec((1, D_IN), lambda i: (0, 0)),
        ],
        out_specs=[
            pl.BlockSpec(memory_space=pltpu.SMEM),
            pl.BlockSpec(memory_space=pltpu.SMEM),
        ],
        out_shape=[
            jax.ShapeDtypeStruct((1, 1), jnp.float32),
            jax.ShapeDtypeStruct((1, 1), jnp.float32),
        ],
        scratch_shapes=[
            pltpu.VMEM((K, D_CODE + 1), jnp.float32),
        ],
    )(x2, W_enc, vectors, v128, W_dec, b2)

    recon = 0.5 * recon_sum[0, 0] / rows + 0.5 * D_IN * jnp.log(2.0 * jnp.pi)
    commit = commit_sum[0, 0] / (rows * D_CODE)
    return recon + _BETA * commit


# speedup vs baseline: 2.4165x; 1.0373x over previous
import jax
import jax.numpy as jnp
from jax import lax
from jax.experimental import pallas as pl
from jax.experimental.pallas import tpu as pltpu

_BETA = 0.25
_ROW_TILE = 4096
_K_TILE = 2048
_HI = 512


def _vq_kernel(x_ref, we_ref, v_ref, v128_ref, wd_ref, b_ref,
               recon_ref, commit_ref, va_ref):
    K, C = v_ref.shape
    n_kt = K // _K_TILE
    rows = x_ref.shape[0]
    lo_width = K // _HI

    @pl.when(pl.program_id(0) == 0)
    def _prep():
        v = v_ref[...]
        va_ref[:, :C] = v + v
        va_ref[:, C:] = -jnp.sum(v * v, axis=1, keepdims=True)

    x = x_ref[...]
    z = jnp.dot(x, we_ref[...], preferred_element_type=jnp.float32)
    z1 = jnp.concatenate([z, jnp.ones((rows, 1), jnp.float32)], axis=1)

    iota_kt = lax.broadcasted_iota(jnp.int32, (1, _K_TILE), 1)

    def scan_body(j, run):
        vat = va_ref[pl.ds(j * _K_TILE, _K_TILE), :]
        score = lax.dot_general(
            z1, vat, (((1,), (1,)), ((), ())),
            preferred_element_type=jnp.float32)
        enc = (K - 1 - j * _K_TILE) - iota_kt
        si = lax.bitcast_convert_type(score, jnp.int32)
        packed = jnp.bitwise_or(jnp.bitwise_and(si, -8192), enc)
        pf = lax.bitcast_convert_type(packed, jnp.float32)
        return jnp.maximum(run, jnp.max(pf, axis=1, keepdims=True))

    run = lax.fori_loop(0, n_kt, scan_body,
                        jnp.full((rows, 1), -jnp.inf, jnp.float32))
    pi = lax.bitcast_convert_type(run, jnp.int32)
    best_idx = (K - 1) - jnp.bitwise_and(pi, 8191)

    lo_shift = lo_width.bit_length() - 1
    hi = best_idx >> lo_shift
    lo = best_idx & (lo_width - 1)
    onehot_hi = (hi == lax.broadcasted_iota(jnp.int32, (1, _HI), 1)
                 ).astype(jnp.float32)
    block = jnp.dot(onehot_hi, v128_ref[...],
                    preferred_element_type=jnp.float32)
    grp = lax.broadcasted_iota(jnp.int32, (1, lo_width * C), 1) // C
    masked = jnp.where(grp == lo, block, 0.0)
    w = lo_width * C
    while w > C:
        w //= 2
        masked = masked[:, :w] + masked[:, w:2 * w]
    q = masked

    z_q = z + (q - z)
    mean = jnp.dot(z_q, wd_ref[...], preferred_element_type=jnp.float32)
    mean = mean + b_ref[...]
    r = x - mean
    dq = q - z
    recon_part = jnp.sum(r * r)
    commit_part = jnp.sum(dq * dq)

    @pl.when(pl.program_id(0) == 0)
    def _init():
        recon_ref[0, 0] = recon_part
        commit_ref[0, 0] = commit_part

    @pl.when(pl.program_id(0) != 0)
    def _acc():
        recon_ref[0, 0] += recon_part
        commit_ref[0, 0] += commit_part


@jax.jit
def kernel(x, W_enc, vectors, W_dec, b_dec):
    B, T, D_IN = x.shape
    K, D_CODE = vectors.shape
    rows = B * T
    x2 = x.reshape(rows, D_IN)
    b2 = b_dec.reshape(1, D_IN)
    v128 = vectors.reshape(_HI, (K // _HI) * D_CODE)
    n_row_tiles = rows // _ROW_TILE

    recon_sum, commit_sum = pl.pallas_call(
        _vq_kernel,
        grid=(n_row_tiles,),
        in_specs=[
            pl.BlockSpec((_ROW_TILE, D_IN), lambda i: (i, 0)),
            pl.BlockSpec((D_IN, D_CODE), lambda i: (0, 0)),
            pl.BlockSpec((K, D_CODE), lambda i: (0, 0)),
            pl.BlockSpec(v128.shape, lambda i: (0, 0)),
            pl.BlockSpec((D_CODE, D_IN), lambda i: (0, 0)),
            pl.BlockSpec((1, D_IN), lambda i: (0, 0)),
        ],
        out_specs=[
            pl.BlockSpec(memory_space=pltpu.SMEM),
            pl.BlockSpec(memory_space=pltpu.SMEM),
        ],
        out_shape=[
            jax.ShapeDtypeStruct((1, 1), jnp.float32),
            jax.ShapeDtypeStruct((1, 1), jnp.float32),
        ],
        scratch_shapes=[
            pltpu.VMEM((K, D_CODE + 1), jnp.float32),
        ],
    )(x2, W_enc, vectors, v128, W_dec, b2)

    recon = 0.5 * recon_sum[0, 0] / rows + 0.5 * D_IN * jnp.log(2.0 * jnp.pi)
    commit = commit_sum[0, 0] / (rows * D_CODE)
    return recon + _BETA * commit


# speedup vs baseline: 2.5695x; 1.0633x over previous
import jax
import jax.numpy as jnp
from jax import lax
from jax.experimental import pallas as pl
from jax.experimental.pallas import tpu as pltpu

_BETA = 0.25
_ROW_TILE = 4096
_K_TILE = 2048
_HI = 512


def _vq_kernel(x_ref, we_ref, v_ref, v128_ref, wd_ref, b_ref,
               recon_ref, commit_ref, va_ref):
    K, C = v_ref.shape
    n_kt = K // _K_TILE
    rows = x_ref.shape[0]
    lo_width = K // _HI

    @pl.when(pl.program_id(0) == 0)
    def _prep():
        v = v_ref[...]
        va_ref[:, :C] = (v + v).astype(jnp.bfloat16)
        va_ref[:, C:] = -jnp.sum(v * v, axis=1,
                                 keepdims=True).astype(jnp.bfloat16)

    x = x_ref[...]
    z = jnp.dot(x, we_ref[...], preferred_element_type=jnp.float32)
    z1 = jnp.concatenate([z, jnp.ones((rows, 1), jnp.float32)],
                         axis=1).astype(jnp.bfloat16)

    iota_kt = lax.broadcasted_iota(jnp.int32, (1, _K_TILE), 1)

    def scan_body(j, run):
        vat = va_ref[pl.ds(j * _K_TILE, _K_TILE), :]
        score = lax.dot_general(
            z1, vat, (((1,), (1,)), ((), ())),
            preferred_element_type=jnp.float32)
        enc = (K - 1 - j * _K_TILE) - iota_kt
        si = lax.bitcast_convert_type(score, jnp.int32)
        packed = jnp.bitwise_or(jnp.bitwise_and(si, -8192), enc)
        pf = lax.bitcast_convert_type(packed, jnp.float32)
        return jnp.maximum(run, jnp.max(pf, axis=1, keepdims=True))

    run = lax.fori_loop(0, n_kt, scan_body,
                        jnp.full((rows, 1), -jnp.inf, jnp.float32))
    pi = lax.bitcast_convert_type(run, jnp.int32)
    best_idx = (K - 1) - jnp.bitwise_and(pi, 8191)

    lo_shift = lo_width.bit_length() - 1
    hi = best_idx >> lo_shift
    lo = best_idx & (lo_width - 1)
    onehot_hi = (hi == lax.broadcasted_iota(jnp.int32, (1, _HI), 1)
                 ).astype(jnp.float32)
    block = jnp.dot(onehot_hi, v128_ref[...],
                    preferred_element_type=jnp.float32)
    grp = lax.broadcasted_iota(jnp.int32, (1, lo_width * C), 1) // C
    masked = jnp.where(grp == lo, block, 0.0)
    w = lo_width * C
    while w > C:
        w //= 2
        masked = masked[:, :w] + masked[:, w:2 * w]
    q = masked

    z_q = z + (q - z)
    mean = jnp.dot(z_q, wd_ref[...], preferred_element_type=jnp.float32)
    mean = mean + b_ref[...]
    r = x - mean
    dq = q - z
    recon_part = jnp.sum(r * r)
    commit_part = jnp.sum(dq * dq)

    @pl.when(pl.program_id(0) == 0)
    def _init():
        recon_ref[0, 0] = recon_part
        commit_ref[0, 0] = commit_part

    @pl.when(pl.program_id(0) != 0)
    def _acc():
        recon_ref[0, 0] += recon_part
        commit_ref[0, 0] += commit_part


@jax.jit
def kernel(x, W_enc, vectors, W_dec, b_dec):
    B, T, D_IN = x.shape
    K, D_CODE = vectors.shape
    rows = B * T
    x2 = x.reshape(rows, D_IN)
    b2 = b_dec.reshape(1, D_IN)
    v128 = vectors.reshape(_HI, (K // _HI) * D_CODE)
    n_row_tiles = rows // _ROW_TILE

    recon_sum, commit_sum = pl.pallas_call(
        _vq_kernel,
        grid=(n_row_tiles,),
        in_specs=[
            pl.BlockSpec((_ROW_TILE, D_IN), lambda i: (i, 0)),
            pl.BlockSpec((D_IN, D_CODE), lambda i: (0, 0)),
            pl.BlockSpec((K, D_CODE), lambda i: (0, 0)),
            pl.BlockSpec(v128.shape, lambda i: (0, 0)),
            pl.BlockSpec((D_CODE, D_IN), lambda i: (0, 0)),
            pl.BlockSpec((1, D_IN), lambda i: (0, 0)),
        ],
        out_specs=[
            pl.BlockSpec(memory_space=pltpu.SMEM),
            pl.BlockSpec(memory_space=pltpu.SMEM),
        ],
        out_shape=[
            jax.ShapeDtypeStruct((1, 1), jnp.float32),
            jax.ShapeDtypeStruct((1, 1), jnp.float32),
        ],
        scratch_shapes=[
            pltpu.VMEM((K, D_CODE + 1), jnp.bfloat16),
        ],
    )(x2, W_enc, vectors, v128, W_dec, b2)

    recon = 0.5 * recon_sum[0, 0] / rows + 0.5 * D_IN * jnp.log(2.0 * jnp.pi)
    commit = commit_sum[0, 0] / (rows * D_CODE)
    return recon + _BETA * commit


# speedup vs baseline: 2.5981x; 1.0112x over previous
import jax
import jax.numpy as jnp
from jax import lax
from jax.experimental import pallas as pl
from jax.experimental.pallas import tpu as pltpu

_BETA = 0.25
_ROW_TILE = 2048
_K_TILE = 4096
_HI = 512


def _vq_kernel(x_ref, we_ref, v_ref, v128_ref, wd_ref, b_ref,
               recon_ref, commit_ref, va_ref):
    K, C = v_ref.shape
    n_kt = K // _K_TILE
    rows = x_ref.shape[0]
    lo_width = K // _HI

    @pl.when(pl.program_id(0) == 0)
    def _prep():
        v = v_ref[...]
        va_ref[:, :C] = (v + v).astype(jnp.bfloat16)
        va_ref[:, C:] = -jnp.sum(v * v, axis=1,
                                 keepdims=True).astype(jnp.bfloat16)

    x = x_ref[...]
    z = jnp.dot(x, we_ref[...], preferred_element_type=jnp.float32)
    z1 = jnp.concatenate([z, jnp.ones((rows, 1), jnp.float32)],
                         axis=1).astype(jnp.bfloat16)

    iota_kt = lax.broadcasted_iota(jnp.int32, (1, _K_TILE), 1)

    def scan_body(j, run):
        vat = va_ref[pl.ds(j * _K_TILE, _K_TILE), :]
        score = lax.dot_general(
            z1, vat, (((1,), (1,)), ((), ())),
            preferred_element_type=jnp.float32)
        enc = (K - 1 - j * _K_TILE) - iota_kt
        si = lax.bitcast_convert_type(score, jnp.int32)
        packed = jnp.bitwise_or(jnp.bitwise_and(si, -8192), enc)
        pf = lax.bitcast_convert_type(packed, jnp.float32)
        return jnp.maximum(run, jnp.max(pf, axis=1, keepdims=True))

    run = lax.fori_loop(0, n_kt, scan_body,
                        jnp.full((rows, 1), -jnp.inf, jnp.float32))
    pi = lax.bitcast_convert_type(run, jnp.int32)
    best_idx = (K - 1) - jnp.bitwise_and(pi, 8191)

    lo_shift = lo_width.bit_length() - 1
    hi = best_idx >> lo_shift
    lo = best_idx & (lo_width - 1)
    onehot_hi = (hi == lax.broadcasted_iota(jnp.int32, (1, _HI), 1)
                 ).astype(jnp.float32)
    block = jnp.dot(onehot_hi, v128_ref[...],
                    preferred_element_type=jnp.float32)
    grp = lax.broadcasted_iota(jnp.int32, (1, lo_width * C), 1) // C
    masked = jnp.where(grp == lo, block, 0.0)
    w = lo_width * C
    while w > C:
        w //= 2
        masked = masked[:, :w] + masked[:, w:2 * w]
    q = masked

    z_q = z + (q - z)
    mean = jnp.dot(z_q, wd_ref[...], preferred_element_type=jnp.float32)
    mean = mean + b_ref[...]
    r = x - mean
    dq = q - z
    recon_part = jnp.sum(r * r)
    commit_part = jnp.sum(dq * dq)

    @pl.when(pl.program_id(0) == 0)
    def _init():
        recon_ref[0, 0] = recon_part
        commit_ref[0, 0] = commit_part

    @pl.when(pl.program_id(0) != 0)
    def _acc():
        recon_ref[0, 0] += recon_part
        commit_ref[0, 0] += commit_part


@jax.jit
def kernel(x, W_enc, vectors, W_dec, b_dec):
    B, T, D_IN = x.shape
    K, D_CODE = vectors.shape
    rows = B * T
    x2 = x.reshape(rows, D_IN)
    b2 = b_dec.reshape(1, D_IN)
    v128 = vectors.reshape(_HI, (K // _HI) * D_CODE)
    n_row_tiles = rows // _ROW_TILE

    recon_sum, commit_sum = pl.pallas_call(
        _vq_kernel,
        grid=(n_row_tiles,),
        in_specs=[
            pl.BlockSpec((_ROW_TILE, D_IN), lambda i: (i, 0)),
            pl.BlockSpec((D_IN, D_CODE), lambda i: (0, 0)),
            pl.BlockSpec((K, D_CODE), lambda i: (0, 0)),
            pl.BlockSpec(v128.shape, lambda i: (0, 0)),
            pl.BlockSpec((D_CODE, D_IN), lambda i: (0, 0)),
            pl.BlockSpec((1, D_IN), lambda i: (0, 0)),
        ],
        out_specs=[
            pl.BlockSpec(memory_space=pltpu.SMEM),
            pl.BlockSpec(memory_space=pltpu.SMEM),
        ],
        out_shape=[
            jax.ShapeDtypeStruct((1, 1), jnp.float32),
            jax.ShapeDtypeStruct((1, 1), jnp.float32),
        ],
        scratch_shapes=[
            pltpu.VMEM((K, D_CODE + 1), jnp.bfloat16),
        ],
    )(x2, W_enc, vectors, v128, W_dec, b2)

    recon = 0.5 * recon_sum[0, 0] / rows + 0.5 * D_IN * jnp.log(2.0 * jnp.pi)
    commit = commit_sum[0, 0] / (rows * D_CODE)
    return recon + _BETA * commit
